# baseline (speedup 1.0000x reference)
import functools

import jax
import jax.numpy as jnp
from jax import lax
from jax.experimental import pallas as pl
from jax.experimental.pallas import tpu as pltpu
from jax.experimental.pallas import tpu_sc as plsc

_NC = 2
_NS = 16
_NW = _NC * _NS
_CB = 16
_QCB = 128
_PITCH = 129
_PSUB = 40


def _sc_transpose(table_t, tail_ev, tail_od):
    d, v = table_t.shape
    nblk = v // _QCB
    tail = v - nblk * _QCB
    base_n = nblk // _NW
    extra = nblk - base_n * _NW
    half = v // 2
    vrow_half = half * d // 128
    mesh = plsc.VectorSubcoreMesh(core_axis_name="c", subcore_axis_name="s")

    @functools.partial(
        pl.kernel,
        out_type=jax.ShapeDtypeStruct((v * d // 128, 128), jnp.float32),
        mesh=mesh,
        scratch_types=[
            pltpu.VMEM((2, d, _QCB), jnp.float32),
            pltpu.VMEM((_PSUB + 32, _PITCH), jnp.float32),
            pltpu.VMEM((_PSUB + 32, _PITCH), jnp.float32),
            pltpu.SemaphoreType.DMA,
            pltpu.SemaphoreType.DMA,
            pltpu.SemaphoreType.DMA,
            pltpu.SemaphoreType.DMA,
        ],
        compiler_params=pltpu.CompilerParams(
            use_tc_tiling_on_sc=True, needs_layout_passes=False
        ),
    )
    def k(tt_hbm, te_hbm, to_hbm, out_hbm, in_v, tr_a, tr_b, si0, si1,
          so0, so1):
        wid = lax.axis_index("s") * _NC + lax.axis_index("c")
        lo = wid * base_n + jnp.minimum(wid, extra)
        n_my = base_n + jnp.where(wid < extra, 1, 0)
        sin = [si0, si1]
        sout = [so0, so1]
        trs = [tr_a, tr_b]

        lane = lax.iota(jnp.int32, 16)
        rowvs = [
            (lane & 1) * _PSUB + (lane >> 2) + 4 * g
            for g in range(_QCB // 16)
        ]
        colbase = ((lane >> 1) & 1) * 64

        def fire_in(blk, buf):
            pltpu.async_copy(
                tt_hbm.at[:, pl.ds(blk * _QCB, _QCB)], in_v.at[buf], sin[buf]
            )

        def drain_in(buf):
            pltpu.make_async_copy(
                tt_hbm.at[:, pl.ds(0, _QCB)], in_v.at[buf], sin[buf]
            ).wait()

        def fire_out(blk, buf):
            vbase = blk * (_QCB // 4)
            pltpu.async_copy(
                trs[buf].at[pl.ds(0, 32), pl.ds(0, 128)],
                out_hbm.at[pl.ds(vbase, 32)],
                sout[buf],
            )
            pltpu.async_copy(
                trs[buf].at[pl.ds(_PSUB, 32), pl.ds(0, 128)],
                out_hbm.at[pl.ds(vrow_half + vbase, 32)],
                sout[buf],
            )

        def drain_out(buf):
            for _ in range(2):
                pltpu.make_async_copy(
                    out_hbm.at[pl.ds(0, 32)],
                    trs[buf].at[pl.ds(0, 32), pl.ds(0, 128)],
                    sout[buf],
                ).wait()

        def transpose(buf):
            src = in_v.at[buf]
            dst = trs[buf]

            def dd_body(dd, carry):
                col = colbase + dd
                for g in range(_QCB // 16):
                    vv = src[dd, pl.ds(16 * g, 16)]
                    plsc.store_scatter(dst, [rowvs[g], col], vv)
                return carry

            lax.fori_loop(0, d, dd_body, 0)

        fire_in(lo, 0)

        def pair_body(p, carry):
            for bb in range(2):
                i = p * 2 + bb

                @pl.when(i < n_my)
                def _():
                    @pl.when(i + 1 < n_my)
                    def _():
                        fire_in(lo + i + 1, 1 - bb)

                    @pl.when(i >= 2)
                    def _():
                        drain_out(bb)

                    drain_in(bb)
                    transpose(bb)
                    fire_out(lo + i, bb)

            return carry

        lax.fori_loop(0, (base_n + 2) // 2, pair_body, 0)
        for bb in range(2):
            @pl.when(n_my > bb)
            def _():
                drain_out(bb)

        if tail:
            tv = tail * d // 256

            @pl.when(wid == _NW - 1)
            def _():
                pltpu.sync_copy(
                    te_hbm, out_hbm.at[pl.ds(vrow_half - tv, tv)]
                )
                pltpu.sync_copy(
                    to_hbm, out_hbm.at[pl.ds(2 * vrow_half - tv, tv)]
                )

    return k(table_t, tail_ev, tail_od)


def _sc_gather(fidx, t1):
    b, l = fidx.shape
    d = t1.shape[1]
    rows_per_w = b // _NW
    nchunk = rows_per_w // _CB
    mesh = plsc.VectorSubcoreMesh(core_axis_name="c", subcore_axis_name="s")

    @functools.partial(
        pl.kernel,
        out_type=jax.ShapeDtypeStruct((b, d), jnp.float32),
        mesh=mesh,
        scratch_types=[
            pltpu.VMEM((2, _CB, l), jnp.int32),
            pltpu.VMEM((2, _CB * l, d), jnp.float32),
            pltpu.VMEM((2, _CB, d), jnp.float32),
            pltpu.SemaphoreType.DMA,
            pltpu.SemaphoreType.DMA,
        ],
        compiler_params=pltpu.CompilerParams(use_tc_tiling_on_sc=False),
    )
    def k(q_hbm, t_hbm, out_hbm, idx_v, rows_v, out_v, sem0, sem1):
        wid = lax.axis_index("s") * _NC + lax.axis_index("c")
        base_row = wid * rows_per_w
        sems = [sem0, sem1]

        beta = [
            (lax.iota(jnp.int32, 16).astype(jnp.float32) + (16.0 * kk - 31.0))
            * (1.0 / 800.0)
            for kk in range(d // 16)
        ]

        def fire(ci, buf):
            row0 = base_row + ci * _CB
            pltpu.sync_copy(q_hbm.at[pl.ds(row0, _CB)], idx_v.at[buf])
            for j in range(_CB):
                pltpu.async_copy(
                    t_hbm.at[idx_v.at[buf].at[j]],
                    rows_v.at[buf].at[pl.ds(j * l, l)],
                    sems[buf],
                )

        def drain(buf):
            pltpu.make_async_copy(
                t_hbm.at[pl.ds(0, _CB * l)], rows_v.at[buf], sems[buf]
            ).wait()

        def compute(ci, buf):
            rows = rows_v.at[buf]
            row0 = base_row + ci * _CB

            def row_body(r, carry2):
                acc0 = [None] * (d // 16)
                acc1 = [None] * (d // 16)
                for li in range(l):
                    alpha = float(li - 24)
                    for kk in range(d // 16):
                        v = rows[r * l + li, pl.ds(16 * kk, 16)]
                        if li == 0:
                            acc0[kk] = v
                            acc1[kk] = alpha * v
                        else:
                            acc0[kk] = acc0[kk] + v
                            if alpha == 1.0:
                                acc1[kk] = acc1[kk] + v
                            elif alpha != 0.0:
                                acc1[kk] = acc1[kk] + alpha * v
                for kk in range(d // 16):
                    out_v[buf, r, pl.ds(16 * kk, 16)] = (
                        acc0[kk] + beta[kk] * acc1[kk]
                    )
                return carry2

            lax.fori_loop(0, _CB, row_body, 0)
            pltpu.sync_copy(out_v.at[buf], out_hbm.at[pl.ds(row0, _CB)])

        fire(0, 0)

        def pair_body(p, carry):
            ci0 = p * 2
            for bb in range(2):
                ci = ci0 + bb
                nxt = ci + 1

                @pl.when(nxt < nchunk)
                def _():
                    fire(nxt, 1 - bb)

                drain(bb)
                compute(ci, bb)
            return carry

        lax.fori_loop(0, nchunk // 2, pair_body, 0)

    return k(fidx, t1)


def kernel(questions, table):
    v, d = table.shape
    q = questions.astype(jnp.int32)
    half = v // 2
    fidx = (q & 1) * half + (q >> 1)
    t1 = jnp.concatenate([table[0::2], table[1::2]], axis=0)
    return _sc_gather(fidx, t1)


# speedup vs baseline: 11.3400x; 11.3400x over previous
import functools

import jax
import jax.numpy as jnp
from jax import lax
from jax.experimental import pallas as pl
from jax.experimental.pallas import tpu as pltpu
from jax.experimental.pallas import tpu_sc as plsc

_NC = 2
_NS = 16
_NW = _NC * _NS
_CB = 8
_QCB = 128
_PITCH = 129
_PSUB = 40


def _sc_transpose(table_t, tail_ev, tail_od):
    d, v = table_t.shape
    nblk = v // _QCB
    tail = v - nblk * _QCB
    base_n = nblk // _NW
    extra = nblk - base_n * _NW
    half = v // 2
    vrow_half = half * d // 128
    mesh = plsc.VectorSubcoreMesh(core_axis_name="c", subcore_axis_name="s")

    @functools.partial(
        pl.kernel,
        out_type=jax.ShapeDtypeStruct((v * d // 128, 128), jnp.float32),
        mesh=mesh,
        scratch_types=[
            pltpu.VMEM((2, d, _QCB), jnp.float32),
            pltpu.VMEM((_PSUB + 32, _PITCH), jnp.float32),
            pltpu.VMEM((_PSUB + 32, _PITCH), jnp.float32),
            pltpu.SemaphoreType.DMA,
            pltpu.SemaphoreType.DMA,
            pltpu.SemaphoreType.DMA,
            pltpu.SemaphoreType.DMA,
        ],
        compiler_params=pltpu.CompilerParams(
            use_tc_tiling_on_sc=True, needs_layout_passes=False
        ),
    )
    def k(tt_hbm, te_hbm, to_hbm, out_hbm, in_v, tr_a, tr_b, si0, si1,
          so0, so1):
        wid = lax.axis_index("s") * _NC + lax.axis_index("c")
        lo = wid * base_n + jnp.minimum(wid, extra)
        n_my = base_n + jnp.where(wid < extra, 1, 0)
        sin = [si0, si1]
        sout = [so0, so1]
        trs = [tr_a, tr_b]

        lane = lax.iota(jnp.int32, 16)
        rowvs = [
            (lane & 1) * _PSUB + (lane >> 2) + 4 * g
            for g in range(_QCB // 16)
        ]
        colbase = ((lane >> 1) & 1) * 64

        def fire_in(blk, buf):
            pltpu.async_copy(
                tt_hbm.at[:, pl.ds(blk * _QCB, _QCB)], in_v.at[buf], sin[buf]
            )

        def drain_in(buf):
            pltpu.make_async_copy(
                tt_hbm.at[:, pl.ds(0, _QCB)], in_v.at[buf], sin[buf]
            ).wait()

        def fire_out(blk, buf):
            vbase = blk * (_QCB // 4)
            pltpu.async_copy(
                trs[buf].at[pl.ds(0, 32), pl.ds(0, 128)],
                out_hbm.at[pl.ds(vbase, 32)],
                sout[buf],
            )
            pltpu.async_copy(
                trs[buf].at[pl.ds(_PSUB, 32), pl.ds(0, 128)],
                out_hbm.at[pl.ds(vrow_half + vbase, 32)],
                sout[buf],
            )

        def drain_out(buf):
            for _ in range(2):
                pltpu.make_async_copy(
                    out_hbm.at[pl.ds(0, 32)],
                    trs[buf].at[pl.ds(0, 32), pl.ds(0, 128)],
                    sout[buf],
                ).wait()

        def transpose(buf):
            src = in_v.at[buf]
            dst = trs[buf]

            def dd_body(dd, carry):
                col = colbase + dd
                for g in range(_QCB // 16):
                    vv = src[dd, pl.ds(16 * g, 16)]
                    plsc.store_scatter(dst, [rowvs[g], col], vv)
                return carry

            lax.fori_loop(0, d, dd_body, 0)

        fire_in(lo, 0)

        def pair_body(p, carry):
            for bb in range(2):
                i = p * 2 + bb

                @pl.when(i < n_my)
                def _():
                    @pl.when(i + 1 < n_my)
                    def _():
                        fire_in(lo + i + 1, 1 - bb)

                    @pl.when(i >= 2)
                    def _():
                        drain_out(bb)

                    drain_in(bb)
                    transpose(bb)
                    fire_out(lo + i, bb)

            return carry

        lax.fori_loop(0, (base_n + 2) // 2, pair_body, 0)
        for bb in range(2):
            @pl.when(n_my > bb)
            def _():
                drain_out(bb)

        if tail:
            tv = tail * d // 256

            @pl.when(wid == _NW - 1)
            def _():
                pltpu.sync_copy(
                    te_hbm, out_hbm.at[pl.ds(vrow_half - tv, tv)]
                )
                pltpu.sync_copy(
                    to_hbm, out_hbm.at[pl.ds(2 * vrow_half - tv, tv)]
                )

    return k(table_t, tail_ev, tail_od)


def _sc_gather(qh, offs, t1):
    b, l = qh.shape
    dp = t1.shape[1]
    d = dp // 2
    rows_per_w = b // _NW
    nchunk = rows_per_w // _CB
    mesh = plsc.VectorSubcoreMesh(core_axis_name="c", subcore_axis_name="s")

    @functools.partial(
        pl.kernel,
        out_type=jax.ShapeDtypeStruct((b, d), jnp.float32),
        mesh=mesh,
        scratch_types=[
            pltpu.VMEM((2, _CB, l), jnp.int32),
            pltpu.VMEM((2, _CB, l), jnp.int32),
            pltpu.VMEM((2, _CB * l, dp), jnp.float32),
            pltpu.VMEM((2, _CB, d), jnp.float32),
            pltpu.SemaphoreType.DMA,
            pltpu.SemaphoreType.DMA,
        ],
        compiler_params=pltpu.CompilerParams(
            use_tc_tiling_on_sc=False, needs_layout_passes=False
        ),
    )
    def k(q_hbm, offs_hbm, t_hbm, out_hbm, idx_v, offs_v, rows_v, out_v,
          sem0, sem1):
        wid = lax.axis_index("s") * _NC + lax.axis_index("c")
        base_row = wid * rows_per_w
        sems = [sem0, sem1]

        beta = [
            (lax.iota(jnp.int32, 16).astype(jnp.float32) + (16.0 * kk - 31.0))
            * (1.0 / 800.0)
            for kk in range(d // 16)
        ]

        def fire(ci, buf):
            row0 = base_row + ci * _CB
            pltpu.sync_copy(q_hbm.at[pl.ds(row0, _CB)], idx_v.at[buf])
            pltpu.sync_copy(offs_hbm.at[pl.ds(row0, _CB)], offs_v.at[buf])
            for j in range(_CB):
                pltpu.async_copy(
                    t_hbm.at[idx_v.at[buf].at[j]],
                    rows_v.at[buf].at[pl.ds(j * l, l)],
                    sems[buf],
                )

        def drain(buf):
            pltpu.make_async_copy(
                t_hbm.at[pl.ds(0, _CB * l)], rows_v.at[buf], sems[buf]
            ).wait()

        lane16 = lax.iota(jnp.int32, 16)
        lmasks = [(lane16 == ln).astype(jnp.float32) for ln in range(16)]

        def compute(ci, buf):
            rows = rows_v.at[buf]
            row0 = base_row + ci * _CB

            def row_body(r, carry2):
                acc0 = [None] * (d // 16)
                acc1 = [None] * (d // 16)
                ovecs = [
                    offs_v[buf, r, pl.ds(s0, 16)].astype(jnp.float32)
                    for s0 in (0, 16, 32, 34)
                ]
                for li in range(l):
                    alpha = float(li - 24)
                    g, ln = (li // 16, li % 16) if li < 48 else (3, li - 34)
                    off_s = jnp.sum(ovecs[g] * lmasks[ln]).astype(jnp.int32)
                    off = pl.multiple_of(off_s, 64)
                    for kk in range(d // 16):
                        v = rows[r * l + li, pl.ds(off + 16 * kk, 16)]
                        if li == 0:
                            acc0[kk] = v
                            acc1[kk] = alpha * v
                        else:
                            acc0[kk] = acc0[kk] + v
                            if alpha == 1.0:
                                acc1[kk] = acc1[kk] + v
                            elif alpha != 0.0:
                                acc1[kk] = acc1[kk] + alpha * v
                for kk in range(d // 16):
                    out_v[buf, r, pl.ds(16 * kk, 16)] = (
                        acc0[kk] + beta[kk] * acc1[kk]
                    )
                return carry2

            lax.fori_loop(0, _CB, row_body, 0)
            pltpu.sync_copy(out_v.at[buf], out_hbm.at[pl.ds(row0, _CB)])

        fire(0, 0)

        def pair_body(p, carry):
            ci0 = p * 2
            for bb in range(2):
                ci = ci0 + bb
                nxt = ci + 1

                @pl.when(nxt < nchunk)
                def _():
                    fire(nxt, 1 - bb)

                drain(bb)
                compute(ci, bb)
            return carry

        lax.fori_loop(0, nchunk // 2, pair_body, 0)

    return k(qh, offs, t1)


def kernel(questions, table):
    v, d = table.shape
    q = questions.astype(jnp.int32)
    qh = q >> 1
    offs = (q & 1) * 64
    return _sc_gather(qh, offs, table.reshape(v // 2, 2 * d))
